# Initial kernel scaffold; baseline (speedup 1.0000x reference)
import jax
import jax.numpy as jnp
from jax.experimental import pallas as pl


def kernel(x, table):
    raise NotImplementedError("write your pallas kernel here")


# speedup vs baseline: 11.8916x; 11.8916x over previous
import functools

import jax
import jax.numpy as jnp
from jax.experimental import pallas as pl
from jax.experimental.pallas import tpu as pltpu

_LEVELS = 1000
_OUT = 1024
_L = 20
_BB = 256


def _ngram_block(x_ref, tbl_ref, out_ref):
    x = x_ref[...]
    idx = jnp.clip(
        jnp.round(x * (_LEVELS - 1)), 0, _LEVELS - 1
    ).astype(jnp.int32)
    tbl = tbl_ref[...]
    lane = jax.lax.broadcasted_iota(jnp.int32, (_BB, _LEVELS), 1)
    acc = None
    for i in range(_L):
        onehot = (idx[:, i][:, None] == lane).astype(jnp.bfloat16)
        g = jax.lax.dot_general(
            onehot,
            tbl,
            (((1,), (0,)), ((), ())),
            preferred_element_type=jnp.float32,
        )
        s = (_L - 1 - i) % _OUT
        if s:
            g = jnp.concatenate([g[:, _OUT - s :], g[:, : _OUT - s]], axis=1)
        acc = g if acc is None else acc * g
    out_ref[...] = acc


@jax.jit
def kernel(x, table):
    B = x.shape[0]
    tbl16 = table.astype(jnp.bfloat16)
    grid = (B // _BB,)
    return pl.pallas_call(
        _ngram_block,
        grid=grid,
        in_specs=[
            pl.BlockSpec((_BB, _L), lambda b: (b, 0)),
            pl.BlockSpec((_LEVELS, _OUT), lambda b: (0, 0)),
        ],
        out_specs=pl.BlockSpec((_BB, _OUT), lambda b: (b, 0)),
        out_shape=jax.ShapeDtypeStruct((B, _OUT), jnp.float32),
    )(x, tbl16)


# speedup vs baseline: 33.9756x; 2.8571x over previous
import functools

import jax
import jax.numpy as jnp
from jax import lax
from jax.experimental import pallas as pl
from jax.experimental.pallas import tpu as pltpu
from jax.experimental.pallas import tpu_sc as plsc

_LEVELS = 1000
_D = 1024
_L = 20
_B = 4096

_NC = 2
_NS = 16
_NW = _NC * _NS
_ROWS_PER_TILE = 32
_SPT = _B // _NW
_SUB = 16
_NSUB = _SPT // _SUB

_MAGIC = 2.0**23
_EXP1 = 0x3F800000
_SIGN = 0x80000000


def _lane_rotate(v, t):
    perm = (lax.iota(jnp.int32, 16) + (16 - t)) & 15
    dnums = lax.GatherDimensionNumbers(
        offset_dims=(),
        collapsed_slice_dims=(0,),
        start_index_map=(0,),
    )
    return lax.gather(
        v,
        perm[:, None],
        dnums,
        (1,),
        indices_are_sorted=False,
        unique_indices=True,
        mode=lax.GatherScatterMode.PROMISE_IN_BOUNDS,
    )


def _rot64(A, B, r):
    if r == 0:
        return A, B
    rr = jnp.uint32(r)
    rl = jnp.uint32(32 - r)
    return (A << rr) | (B >> rl), (B << rr) | (A >> rl)


def _pack_body(tbl_hbm, tp_hbm, tblv, varbuf, sem):
    wid = lax.axis_index("s") * _NC + lax.axis_index("c")
    r0 = jnp.minimum(wid * _ROWS_PER_TILE, _LEVELS - _ROWS_PER_TILE)
    pltpu.sync_copy(tbl_hbm.at[pl.ds(r0, _ROWS_PER_TILE)], tblv)

    zero16f = jnp.zeros((16,), jnp.float32)
    lane = lax.iota(jnp.int32, 16)

    def row_body(ll, carry):
        A = jnp.zeros((16,), jnp.uint32)
        B = jnp.zeros((16,), jnp.uint32)
        for p in range(32):
            bitsA = jnp.where(
                tblv[ll, pl.ds(16 * p, 16)] < zero16f,
                jnp.uint32(1 << p),
                jnp.uint32(0),
            )
            bitsB = jnp.where(
                tblv[ll, pl.ds(16 * (p + 32), 16)] < zero16f,
                jnp.uint32(1 << p),
                jnp.uint32(0),
            )
            A = A | bitsA
            B = B | bitsB
        for i in range(_L):
            s = _L - 1 - i
            q, t = divmod(s, 16)
            if t == 0:
                As, Bs = _rot64(A, B, q)
            else:
                Ag = _lane_rotate(A, t)
                Bg = _lane_rotate(B, t)
                Alo, Blo = _rot64(Ag, Bg, q)
                Ahi, Bhi = _rot64(Ag, Bg, q + 1)
                m = lane < t
                As = jnp.where(m, Ahi, Alo)
                Bs = jnp.where(m, Bhi, Blo)
            varbuf[i, ll, pl.ds(0, 16)] = As
            varbuf[i, ll, pl.ds(16, 16)] = Bs
        return carry

    lax.fori_loop(0, _ROWS_PER_TILE, row_body, 0)

    copies = [
        pltpu.async_copy(
            varbuf.at[i], tp_hbm.at[i, pl.ds(r0, _ROWS_PER_TILE)], sem
        )
        for i in range(_L)
    ]
    for c in copies:
        c.wait()


def _encode_body(xf_hbm, tp_hbm, out_hbm, xv, cidx, rows_v, outbuf, gsem, os0, os1):
    wid = lax.axis_index("s") * _NC + lax.axis_index("c")
    b0 = wid * _SPT
    pltpu.sync_copy(xf_hbm.at[pl.ds(wid * (_SPT * _L), _SPT * _L)], xv)

    lane = lax.iota(jnp.int32, 16)
    nchunks = (_SPT * _L) // 16
    for c in range(nchunks):
        y = xv[pl.ds(16 * c, 16)] * jnp.float32(_LEVELS - 1)
        yr = (y + jnp.float32(_MAGIC)) - jnp.float32(_MAGIC)
        ri = yr.astype(jnp.int32)
        ri = jnp.minimum(jnp.maximum(ri, 0), _LEVELS - 1)
        pos = (lane + (16 * c)) % _L
        ci = pos * _LEVELS + ri
        cidx[c // 8, pl.ds((c % 8) * 16, 16)] = ci

    gathers = [
        pltpu.async_copy(tp_hbm.at[cidx.at[j]], rows_v.at[j], gsem)
        for j in range(_L)
    ]
    for g in gathers:
        g.wait()

    osems = [os0, os1]
    pending = [None, None]
    for sc in range(_NSUB):
        buf = sc % 2
        if pending[buf] is not None:
            pending[buf].wait()
            pending[buf] = None

        def sample_body(bsub, carry, _buf=buf, _sc=sc):
            f0 = (_sc * _SUB + bsub) * _L
            A = jnp.zeros((16,), jnp.uint32)
            B = jnp.zeros((16,), jnp.uint32)
            for t in range(_L):
                f = f0 + t
                j = lax.shift_right_logical(f, 7)
                r = f & 127
                A = A ^ rows_v[j, r, pl.ds(0, 16)]
                B = B ^ rows_v[j, r, pl.ds(16, 16)]
            for p in range(32):
                vA = lax.bitcast_convert_type(
                    ((A << jnp.uint32(31 - p)) & jnp.uint32(_SIGN))
                    | jnp.uint32(_EXP1),
                    jnp.float32,
                )
                vB = lax.bitcast_convert_type(
                    ((B << jnp.uint32(31 - p)) & jnp.uint32(_SIGN))
                    | jnp.uint32(_EXP1),
                    jnp.float32,
                )
                outbuf[_buf, bsub, pl.ds(16 * p, 16)] = vA
                outbuf[_buf, bsub, pl.ds(16 * (p + 32), 16)] = vB
            return carry

        lax.fori_loop(0, _SUB, sample_body, 0)
        pending[buf] = pltpu.async_copy(
            outbuf.at[buf],
            out_hbm.at[pl.ds(b0 + sc * _SUB, _SUB)],
            osems[buf],
        )
    for d in pending:
        if d is not None:
            d.wait()


@jax.jit
def kernel(x, table):
    mesh = plsc.VectorSubcoreMesh(core_axis_name="c", subcore_axis_name="s")

    pack = functools.partial(
        pl.kernel,
        mesh=mesh,
        out_type=jax.ShapeDtypeStruct((_L, _LEVELS, 32), jnp.uint32),
        scratch_types=[
            pltpu.VMEM((_ROWS_PER_TILE, _D), jnp.float32),
            pltpu.VMEM((_L, _ROWS_PER_TILE, 32), jnp.uint32),
            pltpu.SemaphoreType.DMA,
        ],
    )(_pack_body)
    tp = pack(table)

    encode = functools.partial(
        pl.kernel,
        mesh=mesh,
        out_type=jax.ShapeDtypeStruct((_B, _D), jnp.float32),
        scratch_types=[
            pltpu.VMEM((_SPT * _L,), jnp.float32),
            pltpu.VMEM((_L, _SPT * _L // _L), jnp.int32),
            pltpu.VMEM((_L, _SPT, 32), jnp.uint32),
            pltpu.VMEM((2, _SUB, _D), jnp.float32),
            pltpu.SemaphoreType.DMA,
            pltpu.SemaphoreType.DMA,
            pltpu.SemaphoreType.DMA,
        ],
        compiler_params=pltpu.CompilerParams(use_tc_tiling_on_sc=False),
    )(_encode_body)
    return encode(x.reshape(-1), tp.reshape(_L * _LEVELS, 32))


# speedup vs baseline: 36.6115x; 1.0776x over previous
import functools

import jax
import jax.numpy as jnp
from jax import lax
from jax.experimental import pallas as pl
from jax.experimental.pallas import tpu as pltpu
from jax.experimental.pallas import tpu_sc as plsc

_LEVELS = 1000
_D = 1024
_L = 20
_B = 4096

_NC = 2
_NS = 16
_NW = _NC * _NS
_ROWS_PER_TILE = 32
_SPT = _B // _NW
_SUB = 16
_NSUB = _SPT // _SUB

_MAGIC = 2.0**23
_EXP1 = 0x3F800000
_SIGN = 0x80000000


def _lane_rotate(v, t):
    perm = (lax.iota(jnp.int32, 16) + (16 - t)) & 15
    dnums = lax.GatherDimensionNumbers(
        offset_dims=(),
        collapsed_slice_dims=(0,),
        start_index_map=(0,),
    )
    return lax.gather(
        v,
        perm[:, None],
        dnums,
        (1,),
        indices_are_sorted=False,
        unique_indices=True,
        mode=lax.GatherScatterMode.PROMISE_IN_BOUNDS,
    )


def _rot64(A, B, r):
    if r == 0:
        return A, B
    rr = jnp.uint32(r)
    rl = jnp.uint32(32 - r)
    return (A << rr) | (B >> rl), (B << rr) | (A >> rl)


def _pack_body(tbl_hbm, tp_hbm, tblv, varbuf, sem):
    wid = lax.axis_index("s") * _NC + lax.axis_index("c")
    r0 = jnp.minimum(wid * _ROWS_PER_TILE, _LEVELS - _ROWS_PER_TILE)
    pltpu.sync_copy(tbl_hbm.at[pl.ds(r0, _ROWS_PER_TILE)], tblv)

    zero16f = jnp.zeros((16,), jnp.float32)
    lane = lax.iota(jnp.int32, 16)

    def row_body(ll, carry):
        A = jnp.zeros((16,), jnp.uint32)
        B = jnp.zeros((16,), jnp.uint32)
        for p in range(32):
            bitsA = jnp.where(
                tblv[ll, pl.ds(16 * p, 16)] < zero16f,
                jnp.uint32(1 << p),
                jnp.uint32(0),
            )
            bitsB = jnp.where(
                tblv[ll, pl.ds(16 * (p + 32), 16)] < zero16f,
                jnp.uint32(1 << p),
                jnp.uint32(0),
            )
            A = A | bitsA
            B = B | bitsB
        for i in range(_L):
            s = _L - 1 - i
            q, t = divmod(s, 16)
            if t == 0:
                As, Bs = _rot64(A, B, q)
            else:
                Ag = _lane_rotate(A, t)
                Bg = _lane_rotate(B, t)
                Alo, Blo = _rot64(Ag, Bg, q)
                Ahi, Bhi = _rot64(Ag, Bg, q + 1)
                m = lane < t
                As = jnp.where(m, Ahi, Alo)
                Bs = jnp.where(m, Bhi, Blo)
            pr = ll * _L + i
            vrow = lax.shift_right_logical(pr, 2)
            voff = (pr & 3) * 32
            varbuf[vrow, pl.ds(voff, 16)] = As
            varbuf[vrow, pl.ds(voff + 16, 16)] = Bs
        return carry

    lax.fori_loop(0, _ROWS_PER_TILE, row_body, 0)

    pltpu.async_copy(
        varbuf,
        tp_hbm.at[pl.ds(r0 * (_L // 4), _ROWS_PER_TILE * _L // 4)],
        sem,
    ).wait()


def _encode_body(xf_hbm, tp_hbm, out_hbm, xv, cidx, rows_v, outbuf, gsem, os0, os1):
    wid = lax.axis_index("s") * _NC + lax.axis_index("c")
    b0 = wid * _SPT
    pltpu.sync_copy(xf_hbm.at[pl.ds(wid * (_SPT * _L), _SPT * _L)], xv)

    lane = lax.iota(jnp.int32, 16)
    nchunks = (_SPT * _L) // 16
    for c in range(nchunks):
        y = xv[pl.ds(16 * c, 16)] * jnp.float32(_LEVELS - 1)
        yr = (y + jnp.float32(_MAGIC)) - jnp.float32(_MAGIC)
        ri = yr.astype(jnp.int32)
        ri = jnp.minimum(jnp.maximum(ri, 0), _LEVELS - 1)
        pos = (lane + (16 * c)) % _L
        ci = ri * jnp.int32(_L) + pos
        cidx[c // 8, pl.ds((c % 8) * 16, 16)] = ci

    gathers = [
        pltpu.async_copy(tp_hbm.at[cidx.at[j]], rows_v.at[j], gsem)
        for j in range(_L)
    ]
    for g in gathers:
        g.wait()

    osems = [os0, os1]
    pending = [None, None]
    for sc in range(_NSUB):
        buf = sc % 2
        if pending[buf] is not None:
            pending[buf].wait()
            pending[buf] = None

        def sample_body(bsub, carry, _buf=buf, _sc=sc):
            f0 = (_sc * _SUB + bsub) * _L
            A = jnp.zeros((16,), jnp.uint32)
            B = jnp.zeros((16,), jnp.uint32)
            for t in range(_L):
                f = f0 + t
                j = lax.shift_right_logical(f, 7)
                r = f & 127
                A = A ^ rows_v[j, r, pl.ds(0, 16)]
                B = B ^ rows_v[j, r, pl.ds(16, 16)]
            for p in range(32):
                vA = lax.bitcast_convert_type(
                    ((A << jnp.uint32(31 - p)) & jnp.uint32(_SIGN))
                    | jnp.uint32(_EXP1),
                    jnp.float32,
                )
                vB = lax.bitcast_convert_type(
                    ((B << jnp.uint32(31 - p)) & jnp.uint32(_SIGN))
                    | jnp.uint32(_EXP1),
                    jnp.float32,
                )
                outbuf[_buf, bsub, pl.ds(16 * p, 16)] = vA
                outbuf[_buf, bsub, pl.ds(16 * (p + 32), 16)] = vB
            return carry

        lax.fori_loop(0, _SUB, sample_body, 0)
        pending[buf] = pltpu.async_copy(
            outbuf.at[buf],
            out_hbm.at[pl.ds(b0 + sc * _SUB, _SUB)],
            osems[buf],
        )
    for d in pending:
        if d is not None:
            d.wait()


@jax.jit
def kernel(x, table):
    mesh = plsc.VectorSubcoreMesh(core_axis_name="c", subcore_axis_name="s")

    pack = functools.partial(
        pl.kernel,
        mesh=mesh,
        out_type=jax.ShapeDtypeStruct((_L * _LEVELS // 4, 128), jnp.uint32),
        scratch_types=[
            pltpu.VMEM((_ROWS_PER_TILE, _D), jnp.float32),
            pltpu.VMEM((_ROWS_PER_TILE * _L // 4, 128), jnp.uint32),
            pltpu.SemaphoreType.DMA,
        ],
    )(_pack_body)
    tp = pack(table)

    encode = functools.partial(
        pl.kernel,
        mesh=mesh,
        out_type=jax.ShapeDtypeStruct((_B, _D), jnp.float32),
        scratch_types=[
            pltpu.VMEM((_SPT * _L,), jnp.float32),
            pltpu.VMEM((_L, _SPT), jnp.int32),
            pltpu.VMEM((_L, _SPT, 32), jnp.uint32),
            pltpu.VMEM((2, _SUB, _D), jnp.float32),
            pltpu.SemaphoreType.DMA,
            pltpu.SemaphoreType.DMA,
            pltpu.SemaphoreType.DMA,
        ],
        compiler_params=pltpu.CompilerParams(use_tc_tiling_on_sc=False),
    )(_encode_body)
    return encode(x.reshape(-1), tp.reshape(_L * _LEVELS, 32))
